# baseline (speedup 1.0000x reference)
import functools

import jax
import jax.numpy as jnp
from jax import lax
from jax.experimental import pallas as pl
from jax.experimental.pallas import tpu as pltpu
from jax.experimental.pallas import tpu_sc as plsc

CHUNK = 128
NBUF = 4


@functools.lru_cache(maxsize=None)
def _build_gather(B, V, D, dtype_name):
    dtype = jnp.dtype(dtype_name)
    info = plsc.get_sparse_core_info()
    NC, NS = info.num_cores, info.num_subcores
    NW = NC * NS
    assert B % (NW * CHUNK) == 0
    b_per_w = B // NW
    n_ch = b_per_w // CHUNK
    assert n_ch % NBUF == 0
    n_groups = n_ch // NBUF
    mesh = plsc.VectorSubcoreMesh(core_axis_name="c", subcore_axis_name="s")

    @functools.partial(
        pl.kernel,
        mesh=mesh,
        out_type=jax.ShapeDtypeStruct((B, D), dtype),
        compiler_params=pltpu.CompilerParams(use_tc_tiling_on_sc=False),
        scratch_types=[
            pltpu.VMEM((n_ch, CHUNK), jnp.int32),
            pltpu.VMEM((NBUF, CHUNK, D), dtype),
            pltpu.SemaphoreType.DMA,
            pltpu.SemaphoreType.DMA,
        ],
    )
    def k(idx_hbm, table_hbm, out_hbm, idx_v, rows_v, gsem, osem):
        wid = lax.axis_index("s") * NC + lax.axis_index("c")
        base = wid * b_per_w
        pltpu.sync_copy(idx_hbm.at[pl.ds(wid * n_ch, n_ch)], idx_v)

        def gather(j, b):
            pltpu.async_copy(table_hbm.at[idx_v.at[j]], rows_v.at[b], gsem)

        def wait_gather(b):
            pltpu.make_async_copy(
                table_hbm.at[idx_v.at[0]], rows_v.at[b], gsem
            ).wait()

        def put(j, b):
            pltpu.async_copy(
                rows_v.at[b], out_hbm.at[pl.ds(base + j * CHUNK, CHUNK)], osem
            )

        def wait_put(j, b):
            pltpu.make_async_copy(
                rows_v.at[b], out_hbm.at[pl.ds(base + j * CHUNK, CHUNK)], osem
            ).wait()

        for b in range(NBUF):
            gather(b, b)

        def body(g, _):
            j0 = g * NBUF
            for b in range(NBUF):
                wait_gather(b)
            for b in range(NBUF):
                put(j0 + b, b)
            @pl.when(g + 1 < n_groups)
            def _():
                for b in range(NBUF):
                    wait_put(j0 + b, b)
                for b in range(NBUF):
                    gather(j0 + NBUF + b, b)
            return 0

        lax.fori_loop(0, n_groups, body, 0)
        for b in range(NBUF):
            wait_put((n_groups - 1) * NBUF + b, b)

    return k


@jax.jit
def kernel(input_sequences, weight):
    batch, hist = input_sequences.shape
    vocab, dim = weight.shape
    idx = input_sequences.reshape(-1).astype(jnp.int32).reshape(-1, CHUNK)
    fn = _build_gather(batch * hist, vocab, dim, weight.dtype.name)
    out = fn(idx, weight)
    return out.reshape(batch, hist, dim)


# speedup vs baseline: 1.0247x; 1.0247x over previous
import functools

import jax
import jax.numpy as jnp
from jax import lax
from jax.experimental import pallas as pl
from jax.experimental.pallas import tpu as pltpu
from jax.experimental.pallas import tpu_sc as plsc

S = 512


@functools.lru_cache(maxsize=None)
def _build_gather(B, V, D, dtype_name):
    dtype = jnp.dtype(dtype_name)
    info = plsc.get_sparse_core_info()
    NC, NS = info.num_cores, info.num_subcores
    NW = NC * NS
    assert B % (NW * S) == 0
    b_per_w = B // NW
    n_sc = b_per_w // S
    assert n_sc % 2 == 0
    mesh = plsc.VectorSubcoreMesh(core_axis_name="c", subcore_axis_name="s")

    @functools.partial(
        pl.kernel,
        mesh=mesh,
        out_type=jax.ShapeDtypeStruct((B, D), dtype),
        compiler_params=pltpu.CompilerParams(use_tc_tiling_on_sc=False),
        scratch_types=[
            pltpu.VMEM((b_per_w,), jnp.int32),
            pltpu.VMEM((S, D), dtype),
            pltpu.VMEM((S, D), dtype),
            pltpu.SemaphoreType.DMA,
            pltpu.SemaphoreType.DMA,
            pltpu.SemaphoreType.DMA,
            pltpu.SemaphoreType.DMA,
        ],
    )
    def k(idx_hbm, table_hbm, out_hbm, idx_v, rows_a, rows_b,
          gsem_a, gsem_b, osem_a, osem_b):
        wid = lax.axis_index("s") * NC + lax.axis_index("c")
        r0 = wid * b_per_w
        pltpu.sync_copy(idx_hbm.at[pl.ds(r0, b_per_w)], idx_v)

        def gather(sc, rows, sem):
            pltpu.async_copy(table_hbm.at[idx_v.at[pl.ds(sc * S, S)]],
                             rows, sem)

        def wait_gather(rows, sem):
            pltpu.make_async_copy(table_hbm.at[idx_v.at[pl.ds(0, S)]],
                                  rows, sem).wait()

        def put(sc, rows, sem):
            pltpu.async_copy(rows, out_hbm.at[pl.ds(r0 + sc * S, S)], sem)

        def wait_put(sc, rows, sem):
            pltpu.make_async_copy(rows, out_hbm.at[pl.ds(r0 + sc * S, S)],
                                  sem).wait()

        gather(0, rows_a, gsem_a)
        gather(1, rows_b, gsem_b)

        def body(p, _):
            sc_a = 2 * p
            sc_b = 2 * p + 1
            wait_gather(rows_a, gsem_a)
            put(sc_a, rows_a, osem_a)
            wait_put(sc_a, rows_a, osem_a)

            @pl.when(sc_a + 2 < n_sc)
            def _():
                gather(sc_a + 2, rows_a, gsem_a)

            wait_gather(rows_b, gsem_b)
            put(sc_b, rows_b, osem_b)
            wait_put(sc_b, rows_b, osem_b)

            @pl.when(sc_b + 2 < n_sc)
            def _():
                gather(sc_b + 2, rows_b, gsem_b)

            return 0

        lax.fori_loop(0, n_sc // 2, body, 0)

    return k


@jax.jit
def kernel(input_sequences, weight):
    batch, hist = input_sequences.shape
    vocab, dim = weight.shape
    idx = input_sequences.reshape(-1).astype(jnp.int32)
    fn = _build_gather(batch * hist, vocab, dim, weight.dtype.name)
    out = fn(idx, weight)
    return out.reshape(batch, hist, dim)
